# Initial kernel scaffold; baseline (speedup 1.0000x reference)
import jax
import jax.numpy as jnp
from jax.experimental import pallas as pl


def kernel(x):
    raise NotImplementedError("write your pallas kernel here")


# speedup vs baseline: 6.5762x; 6.5762x over previous
import functools
import math

import jax
import jax.numpy as jnp
from jax import lax
from jax.experimental import pallas as pl
from jax.experimental.pallas import tpu as pltpu
from jax.experimental.pallas import tpu_sc as plsc

_B = 128
_E = 32768
_K = math.ceil(0.05 * _E)
_L = 16
_NCHUNK = _E // _L
_H12 = 4096
_H8 = 256
_NTILES = 32
_ROWS_PER_TILE = _B // _NTILES
_MININT = -2147483648
_ONEBITS = 0x3F800000


def _scan_hist(h_ref, nbins, limit, ids16):
    def body(i, carry):
        run, bestj, best_c = carry
        h = h_ref[pl.ds(i * _L, _L)]
        c = plsc.cumsum(h) + run
        ok = c <= limit
        ids = ids16 + i * _L
        cand = jnp.where(ok, ids, jnp.int32(-1))
        cand_c = jnp.where(ok, c, jnp.int32(0))
        bestj = jnp.maximum(bestj, jnp.max(cand))
        best_c = jnp.maximum(best_c, jnp.max(cand_c))
        return jnp.max(c), bestj, best_c

    init = (jnp.int32(0), jnp.int32(-1), jnp.int32(0))
    _, bestj, best_c = lax.fori_loop(0, nbins // _L, body, init)
    return bestj + 1, best_c


def _sortable_key(b):
    return b ^ ((b >> 31) | _MININT)


def _tile_body(x_hbm, out_hbm, row_v, h1_v, h2_v, h3_v):
    cid = lax.axis_index("c")
    sid = lax.axis_index("s")
    wid = sid * 2 + cid

    zero16 = jnp.zeros((_L,), jnp.int32)
    one16 = jnp.ones((_L,), jnp.int32)
    ids16 = lax.iota(jnp.int32, _L)
    limit1 = jnp.int32(_E - _K)

    for rr in range(_ROWS_PER_TILE):
        row = wid * _ROWS_PER_TILE + rr
        pltpu.sync_copy(x_hbm.at[pl.ds(row * _E, _E)], row_v)

        def zero12(i, carry):
            h1_v[pl.ds(i * _L, _L)] = zero16
            h2_v[pl.ds(i * _L, _L)] = zero16
            return carry

        lax.fori_loop(0, _H12 // _L, zero12, jnp.int32(0))

        def zero8(i, carry):
            h3_v[pl.ds(i * _L, _L)] = zero16
            return carry

        lax.fori_loop(0, _H8 // _L, zero8, jnp.int32(0))

        def pass1(i, carry):
            u = _sortable_key(row_v[pl.ds(i * _L, _L)])
            f1 = lax.shift_right_logical(u, 20)
            plsc.addupdate_scatter(h1_v, [f1], one16)
            return carry

        lax.fori_loop(0, _NCHUNK, pass1, jnp.int32(0))
        t1, bc1 = _scan_hist(h1_v, _H12, limit1, ids16)
        limit2 = limit1 - bc1

        def pass2(i, carry):
            u = _sortable_key(row_v[pl.ds(i * _L, _L)])
            f1 = lax.shift_right_logical(u, 20)
            f2 = lax.shift_right_logical(u, 8) & jnp.int32(0xFFF)
            plsc.addupdate_scatter(h2_v, [f2], one16, mask=f1 == t1)
            return carry

        lax.fori_loop(0, _NCHUNK, pass2, jnp.int32(0))
        t2, bc2 = _scan_hist(h2_v, _H12, limit2, ids16)
        limit3 = limit2 - bc2
        pref24 = (t1 << 12) | t2

        def pass3(i, carry):
            u = _sortable_key(row_v[pl.ds(i * _L, _L)])
            hi24 = lax.shift_right_logical(u, 8)
            f3 = u & jnp.int32(0xFF)
            plsc.addupdate_scatter(h3_v, [f3], one16, mask=hi24 == pref24)
            return carry

        lax.fori_loop(0, _NCHUNK, pass3, jnp.int32(0))
        t3, bc3 = _scan_hist(h3_v, _H8, limit3, ids16)
        limit4 = limit3 - bc3

        def pick_n3(i, acc):
            h = h3_v[pl.ds(i * _L, _L)]
            ids = ids16 + i * _L
            return jnp.maximum(acc, jnp.max(jnp.where(ids == t3, h, zero16)))

        n3 = lax.fori_loop(0, _H8 // _L, pick_n3, jnp.int32(0))
        need_eq = n3 - limit4
        s_thr = ((t1 << 20) | (t2 << 8) | t3) ^ _MININT

        def emit(i, run_eq):
            u = _sortable_key(row_v[pl.ds(i * _L, _L)])
            s = u ^ _MININT
            gt = s > s_thr
            eq = s == s_thr
            cum = plsc.cumsum(eq.astype(jnp.int32)) + run_eq
            take = eq & (cum <= need_eq)
            row_v[pl.ds(i * _L, _L)] = jnp.where(
                gt | take, jnp.int32(_ONEBITS), jnp.int32(0))
            return jnp.max(cum)

        lax.fori_loop(0, _NCHUNK, emit, jnp.int32(0))
        pltpu.sync_copy(row_v, out_hbm.at[pl.ds(row * _E, _E)])


def kernel(x):
    mesh = plsc.VectorSubcoreMesh(core_axis_name="c", subcore_axis_name="s")
    run = pl.kernel(
        _tile_body,
        out_type=jax.ShapeDtypeStruct((_B * _E,), jnp.int32),
        mesh=mesh,
        compiler_params=pltpu.CompilerParams(needs_layout_passes=False),
        scratch_types=[
            pltpu.VMEM((_E,), jnp.int32),
            pltpu.VMEM((_H12,), jnp.int32),
            pltpu.VMEM((_H12,), jnp.int32),
            pltpu.VMEM((_H8,), jnp.int32),
        ],
    )
    xi = lax.bitcast_convert_type(x.reshape(-1), jnp.int32)
    out = run(xi)
    return lax.bitcast_convert_type(out, jnp.float32).reshape(_B, _E)


# speedup vs baseline: 10.6058x; 1.6127x over previous
import functools
import math

import jax
import jax.numpy as jnp
from jax import lax
from jax.experimental import pallas as pl
from jax.experimental.pallas import tpu as pltpu
from jax.experimental.pallas import tpu_sc as plsc

_B = 128
_E = 32768
_K = math.ceil(0.05 * _E)
_L = 16
_NCHUNK = _E // _L
_H12 = 4096
_H8 = 256
_NTILES = 32
_ROWS_PER_TILE = _B // _NTILES
_UNROLL = 4
_MININT = -2147483648
_ONEBITS = 0x3F800000


def _scan_hist(h_ref, nbins, limit):
    zeros = jnp.zeros((_L,), jnp.int32)

    def body(i, carry):
        run, cnt, mx = carry
        h = h_ref[pl.ds(i * _L, _L)]
        c = plsc.cumsum(h) + run
        ok = c <= limit
        cnt = cnt + plsc.all_reduce_population_count(ok)
        mx = jnp.maximum(mx, jnp.where(ok, c, zeros))
        return run + jnp.sum(h), cnt, mx

    _, cnt, mx = lax.fori_loop(0, nbins // _L, body, (zeros, zeros, zeros))
    return jnp.max(cnt), jnp.max(mx)


def _tile_body(x_hbm, out_hbm, row_v, h1_v, h2_v, h3_v):
    cid = lax.axis_index("c")
    sid = lax.axis_index("s")
    wid = sid * 2 + cid

    zero16 = jnp.zeros((_L,), jnp.int32)
    one16 = jnp.ones((_L,), jnp.int32)
    ids16 = lax.iota(jnp.int32, _L)
    limit1 = jnp.int32(_E - _K)

    for rr in range(_ROWS_PER_TILE):
        row = wid * _ROWS_PER_TILE + rr
        pltpu.sync_copy(x_hbm.at[pl.ds(row * _E, _E)], row_v)

        def zero12(i, carry):
            for j in range(_UNROLL):
                h1_v[pl.ds((i * _UNROLL + j) * _L, _L)] = zero16
                h2_v[pl.ds((i * _UNROLL + j) * _L, _L)] = zero16
            return carry

        lax.fori_loop(0, _H12 // _L // _UNROLL, zero12, jnp.int32(0))

        def zero8(i, carry):
            h3_v[pl.ds(i * _L, _L)] = zero16
            return carry

        lax.fori_loop(0, _H8 // _L, zero8, jnp.int32(0))

        def pass1(i, carry):
            for j in range(_UNROLL):
                sl = pl.ds((i * _UNROLL + j) * _L, _L)
                b = row_v[sl]
                u = b ^ ((b >> 31) | _MININT)
                row_v[sl] = u
                f1 = lax.shift_right_logical(u, 20)
                plsc.addupdate_scatter(h1_v, [f1], one16)
            return carry

        lax.fori_loop(0, _NCHUNK // _UNROLL, pass1, jnp.int32(0))
        t1, bc1 = _scan_hist(h1_v, _H12, limit1)
        limit2 = limit1 - bc1

        def pass2(i, carry):
            for j in range(_UNROLL):
                u = row_v[pl.ds((i * _UNROLL + j) * _L, _L)]
                f1 = lax.shift_right_logical(u, 20)
                f2 = lax.shift_right_logical(u, 8) & jnp.int32(0xFFF)
                plsc.addupdate_scatter(h2_v, [f2], one16, mask=f1 == t1)
            return carry

        lax.fori_loop(0, _NCHUNK // _UNROLL, pass2, jnp.int32(0))
        t2, bc2 = _scan_hist(h2_v, _H12, limit2)
        limit3 = limit2 - bc2
        pref24 = (t1 << 12) | t2

        def pass3(i, carry):
            for j in range(_UNROLL):
                u = row_v[pl.ds((i * _UNROLL + j) * _L, _L)]
                hi24 = lax.shift_right_logical(u, 8)
                f3 = u & jnp.int32(0xFF)
                plsc.addupdate_scatter(h3_v, [f3], one16, mask=hi24 == pref24)
            return carry

        lax.fori_loop(0, _NCHUNK // _UNROLL, pass3, jnp.int32(0))
        t3, bc3 = _scan_hist(h3_v, _H8, limit3)
        limit4 = limit3 - bc3
        s_thr = ((t1 << 20) | (t2 << 8) | t3) ^ _MININT

        @pl.when(limit4 == 0)
        def _emit_fast():
            def emit(i, carry):
                for j in range(_UNROLL):
                    sl = pl.ds((i * _UNROLL + j) * _L, _L)
                    s = row_v[sl] ^ _MININT
                    row_v[sl] = jnp.where(
                        s >= s_thr, jnp.int32(_ONEBITS), zero16)
                return carry

            lax.fori_loop(0, _NCHUNK // _UNROLL, emit, jnp.int32(0))

        @pl.when(limit4 > 0)
        def _emit_ties():
            def pick_n3(i, acc):
                h = h3_v[pl.ds(i * _L, _L)]
                ids = ids16 + i * _L
                return jnp.maximum(acc, jnp.where(ids == t3, h, zero16))

            n3v = lax.fori_loop(0, _H8 // _L, pick_n3, zero16)
            need_eq = jnp.max(n3v) - limit4

            def emit(i, run_eq):
                sl = pl.ds(i * _L, _L)
                s = row_v[sl] ^ _MININT
                gt = s > s_thr
                eq = s == s_thr
                cum = plsc.cumsum(eq.astype(jnp.int32)) + run_eq
                take = eq & (cum <= need_eq)
                row_v[sl] = jnp.where(
                    gt | take, jnp.int32(_ONEBITS), zero16)
                return jnp.max(cum)

            lax.fori_loop(0, _NCHUNK, emit, jnp.int32(0))

        pltpu.sync_copy(row_v, out_hbm.at[pl.ds(row * _E, _E)])


def kernel(x):
    mesh = plsc.VectorSubcoreMesh(core_axis_name="c", subcore_axis_name="s")
    run = pl.kernel(
        _tile_body,
        out_type=jax.ShapeDtypeStruct((_B * _E,), jnp.int32),
        mesh=mesh,
        compiler_params=pltpu.CompilerParams(needs_layout_passes=False),
        scratch_types=[
            pltpu.VMEM((_E,), jnp.int32),
            pltpu.VMEM((_H12,), jnp.int32),
            pltpu.VMEM((_H12,), jnp.int32),
            pltpu.VMEM((_H8,), jnp.int32),
        ],
    )
    xi = lax.bitcast_convert_type(x.reshape(-1), jnp.int32)
    out = run(xi)
    return lax.bitcast_convert_type(out, jnp.float32).reshape(_B, _E)


# speedup vs baseline: 11.5981x; 1.0936x over previous
import functools
import math

import jax
import jax.numpy as jnp
from jax import lax
from jax.experimental import pallas as pl
from jax.experimental.pallas import tpu as pltpu
from jax.experimental.pallas import tpu_sc as plsc

_B = 128
_E = 32768
_K = math.ceil(0.05 * _E)
_L = 16
_NCHUNK = _E // _L
_H12 = 4096
_H8 = 256
_NTILES = 32
_ROWS_PER_TILE = _B // _NTILES
_UNROLL = 4
_MININT = -2147483648
_ONEBITS = 0x3F800000


def _scan_hist(h_ref, nbins, limit, unroll=1):
    zeros = jnp.zeros((_L,), jnp.int32)

    def body(i, carry):
        run, cnt, mx = carry
        for j in range(unroll):
            h = h_ref[pl.ds((i * unroll + j) * _L, _L)]
            c = plsc.cumsum(h) + run
            ok = c <= limit
            cnt = cnt + plsc.all_reduce_population_count(ok)
            mx = jnp.maximum(mx, jnp.where(ok, c, zeros))
            run = run + jnp.sum(h)
        return run, cnt, mx

    _, cnt, mx = lax.fori_loop(
        0, nbins // _L // unroll, body, (zeros, zeros, zeros))
    return jnp.max(cnt), jnp.max(mx)


def _tile_body(x_hbm, out_hbm, row_v, buf_v, h1_v, h2_v, h3_v, h4_v):
    cid = lax.axis_index("c")
    sid = lax.axis_index("s")
    wid = sid * 2 + cid

    zero16 = jnp.zeros((_L,), jnp.int32)
    one16 = jnp.ones((_L,), jnp.int32)
    ids16 = lax.iota(jnp.int32, _L)
    limit1 = jnp.int32(_E - _K)

    for rr in range(_ROWS_PER_TILE):
        row = wid * _ROWS_PER_TILE + rr
        pltpu.sync_copy(x_hbm.at[pl.ds(row * _E, _E)], row_v)

        def zero12(i, carry):
            for j in range(_UNROLL):
                h1_v[pl.ds((i * _UNROLL + j) * _L, _L)] = zero16
            return carry

        lax.fori_loop(0, _H12 // _L // _UNROLL, zero12, jnp.int32(0))

        def zero8(i, carry):
            h2_v[pl.ds(i * _L, _L)] = zero16
            h3_v[pl.ds(i * _L, _L)] = zero16
            return carry

        lax.fori_loop(0, _H8 // _L, zero8, jnp.int32(0))
        h4_v[pl.ds(0, _L)] = zero16

        def pass1(i, carry):
            for j in range(_UNROLL):
                sl = pl.ds((i * _UNROLL + j) * _L, _L)
                b = row_v[sl]
                u = b ^ ((b >> 31) | _MININT)
                row_v[sl] = u
                f1 = lax.shift_right_logical(u, 20)
                plsc.addupdate_scatter(h1_v, [f1], one16)
            return carry

        lax.fori_loop(0, _NCHUNK // _UNROLL, pass1, jnp.int32(0))
        t1, bc1 = _scan_hist(h1_v, _H12, limit1, unroll=_UNROLL)
        limit2 = limit1 - bc1

        def compact(i, off):
            for j in range(_UNROLL):
                u = row_v[pl.ds((i * _UNROLL + j) * _L, _L)]
                msk = lax.shift_right_logical(u, 20) == t1
                mi = msk.astype(jnp.int32)
                pos = off + plsc.cumsum(mi) - mi
                plsc.store_scatter(buf_v, [pos], u, mask=msk)
                off = off + plsc.all_reduce_population_count(msk)
            return off

        off = lax.fori_loop(0, _NCHUNK // _UNROLL, compact, zero16)
        rem = jnp.max(off)

        def refine(h_ref, nbins, shift, fmask, limit, rem):
            trips = (rem + _L - 1) // _L

            def hist(i, carry):
                u = buf_v[pl.ds(i * _L, _L)]
                valid = ids16 + i * _L < rem
                f = lax.shift_right_logical(u, shift) & jnp.int32(fmask)
                plsc.addupdate_scatter(h_ref, [f], one16, mask=valid)
                return carry

            lax.fori_loop(0, trips, hist, jnp.int32(0))
            t, bc = _scan_hist(h_ref, nbins, limit)

            def recompact(i, off):
                u = buf_v[pl.ds(i * _L, _L)]
                valid = ids16 + i * _L < rem
                f = lax.shift_right_logical(u, shift) & jnp.int32(fmask)
                msk = valid & (f == t)
                mi = msk.astype(jnp.int32)
                pos = off + plsc.cumsum(mi) - mi
                plsc.store_scatter(buf_v, [pos], u, mask=msk)
                return off + plsc.all_reduce_population_count(msk)

            off = lax.fori_loop(0, trips, recompact, zero16)
            return t, limit - bc, jnp.max(off)

        t2, limit3, rem = refine(h2_v, _H8, 12, 0xFF, limit2, rem)
        t3, limit4, rem = refine(h3_v, _H8, 4, 0xFF, limit3, rem)

        def hist4(i, carry):
            u = buf_v[pl.ds(i * _L, _L)]
            valid = ids16 + i * _L < rem
            f = u & jnp.int32(0xF)
            plsc.addupdate_scatter(h4_v, [f], one16, mask=valid)
            return carry

        lax.fori_loop(0, (rem + _L - 1) // _L, hist4, jnp.int32(0))
        t4, bc4 = _scan_hist(h4_v, _L, limit4)
        limit5 = limit4 - bc4

        h4 = h4_v[pl.ds(0, _L)]
        n4 = jnp.max(jnp.where(ids16 == t4, h4, zero16))
        need_eq = n4 - limit5
        s_thr = ((t1 << 20) | (t2 << 12) | (t3 << 4) | t4) ^ _MININT

        @pl.when(limit5 == 0)
        def _emit_fast():
            def emit(i, carry):
                for j in range(_UNROLL):
                    sl = pl.ds((i * _UNROLL + j) * _L, _L)
                    s = row_v[sl] ^ _MININT
                    row_v[sl] = jnp.where(
                        s >= s_thr, jnp.int32(_ONEBITS), zero16)
                return carry

            lax.fori_loop(0, _NCHUNK // _UNROLL, emit, jnp.int32(0))

        @pl.when(limit5 > 0)
        def _emit_ties():
            def emit(i, run_eq):
                sl = pl.ds(i * _L, _L)
                s = row_v[sl] ^ _MININT
                gt = s > s_thr
                eq = s == s_thr
                cum = plsc.cumsum(eq.astype(jnp.int32)) + run_eq
                take = eq & (cum <= need_eq)
                row_v[sl] = jnp.where(
                    gt | take, jnp.int32(_ONEBITS), zero16)
                return jnp.max(cum)

            lax.fori_loop(0, _NCHUNK, emit, jnp.int32(0))

        pltpu.sync_copy(row_v, out_hbm.at[pl.ds(row * _E, _E)])


def kernel(x):
    mesh = plsc.VectorSubcoreMesh(core_axis_name="c", subcore_axis_name="s")
    run = pl.kernel(
        _tile_body,
        out_type=jax.ShapeDtypeStruct((_B * _E,), jnp.int32),
        mesh=mesh,
        compiler_params=pltpu.CompilerParams(needs_layout_passes=False),
        scratch_types=[
            pltpu.VMEM((_E,), jnp.int32),
            pltpu.VMEM((_E,), jnp.int32),
            pltpu.VMEM((_H12,), jnp.int32),
            pltpu.VMEM((_H8,), jnp.int32),
            pltpu.VMEM((_H8,), jnp.int32),
            pltpu.VMEM((_L,), jnp.int32),
        ],
    )
    xi = lax.bitcast_convert_type(x.reshape(-1), jnp.int32)
    out = run(xi)
    return lax.bitcast_convert_type(out, jnp.float32).reshape(_B, _E)
